# baseline (speedup 1.0000x reference)
import jax
import jax.numpy as jnp
from jax.experimental import pallas as pl
from jax.experimental.pallas import tpu as pltpu

_TN = 2048


def _chamfer_body(xp_ref, y_ref, out_ref, miny_ref):
    b = pl.program_id(0)
    i = pl.program_id(1)
    nb = pl.num_programs(1)

    xb = xp_ref[0]
    yv = y_ref[0]

    f32 = jnp.float32
    bf16 = jnp.bfloat16
    x2 = (xb[:, 0:1] * xb[:, 0:1] + xb[:, 1:2] * xb[:, 1:2]
          + xb[:, 2:3] * xb[:, 2:3])
    y2 = (yv[0:1, :] * yv[0:1, :] + yv[1:2, :] * yv[1:2, :]
          + yv[2:3, :] * yv[2:3, :])
    x2_hi = x2.astype(bf16)
    x2_lo = (x2 - x2_hi.astype(f32)).astype(bf16)
    y2_hi = y2.astype(bf16)
    y2_lo = (y2 - y2_hi.astype(f32)).astype(bf16)
    ones_c = jnp.ones_like(x2, dtype=bf16)
    ones_r = jnp.ones_like(y2, dtype=bf16)
    zero_c = jnp.zeros_like(x2, dtype=bf16)
    zero_r = jnp.zeros_like(y2, dtype=bf16)
    a_mat = jnp.concatenate(
        [x2_hi, x2_lo, ones_c, ones_c,
         (-2.0 * xb[:, 0:1]).astype(bf16),
         (-2.0 * xb[:, 1:2]).astype(bf16),
         (-2.0 * xb[:, 2:3]).astype(bf16), zero_c], axis=1)
    b_mat = jnp.concatenate(
        [ones_r, ones_r, y2_hi, y2_lo,
         yv[0:1, :].astype(bf16), yv[1:2, :].astype(bf16),
         yv[2:3, :].astype(bf16), zero_r], axis=0)
    d32 = jax.lax.dot_general(
        a_mat, b_mat, (((1,), (0,)), ((), ())),
        preferred_element_type=f32)
    d = d32.astype(bf16)

    s_x = jnp.sum(jnp.maximum(jnp.min(d, axis=1).astype(f32), 0.0))
    tile_miny = jnp.min(d, axis=0, keepdims=True)

    new_miny = jnp.where(i == 0, tile_miny,
                         jnp.minimum(miny_ref[...], tile_miny))
    miny_ref[...] = new_miny

    inc = s_x + jnp.where(
        i == nb - 1,
        jnp.sum(jnp.maximum(new_miny.astype(jnp.float32), 0.0)), 0.0)
    first = jnp.logical_and(b == 0, i == 0)
    out_ref[0, 0] = jnp.where(first, 0.0, out_ref[0, 0]) + inc


def kernel(x, y):
    B, D, N = x.shape
    M = y.shape[2]
    xp = jnp.transpose(x, (0, 2, 1))

    nb = N // _TN
    out = pl.pallas_call(
        _chamfer_body,
        grid=(B, nb),
        in_specs=[
            pl.BlockSpec((1, _TN, D), lambda b, i: (b, i, 0)),
            pl.BlockSpec((1, D, M), lambda b, i: (b, 0, 0)),
        ],
        out_specs=pl.BlockSpec((1, 1), lambda b, i: (0, 0),
                               memory_space=pltpu.SMEM),
        out_shape=jax.ShapeDtypeStruct((1, 1), jnp.float32),
        scratch_shapes=[pltpu.VMEM((1, M), jnp.bfloat16)],
        compiler_params=pltpu.CompilerParams(
            dimension_semantics=("arbitrary", "arbitrary")),
    )(xp, y)

    return out[0, 0] / jnp.float32(B * N)


# speedup vs baseline: 1.2278x; 1.2278x over previous
import jax
import jax.numpy as jnp
from jax.experimental import pallas as pl
from jax.experimental.pallas import tpu as pltpu

_TN = 2048


def _chamfer_body(x_ref, y_ref, out_ref, miny_ref):
    b = pl.program_id(0)
    i = pl.program_id(1)
    nb = pl.num_programs(1)

    xv = x_ref[0]
    yv = y_ref[0]

    f32 = jnp.float32
    bf16 = jnp.bfloat16

    def _rows(v):
        v2 = (v[0:1, :] * v[0:1, :] + v[1:2, :] * v[1:2, :]
              + v[2:3, :] * v[2:3, :])
        v2_hi = v2.astype(bf16)
        v2_lo = (v2 - v2_hi.astype(f32)).astype(bf16)
        ones = jnp.ones_like(v2, dtype=bf16)
        zero = jnp.zeros_like(v2, dtype=bf16)
        return v2_hi, v2_lo, ones, zero

    x2_hi, x2_lo, ones_x, zero_x = _rows(xv)
    y2_hi, y2_lo, ones_y, zero_y = _rows(yv)
    a_t = jnp.concatenate(
        [x2_hi, x2_lo, ones_x, ones_x,
         (-2.0 * xv[0:1, :]).astype(bf16),
         (-2.0 * xv[1:2, :]).astype(bf16),
         (-2.0 * xv[2:3, :]).astype(bf16), zero_x], axis=0)
    b_mat = jnp.concatenate(
        [ones_y, ones_y, y2_hi, y2_lo,
         yv[0:1, :].astype(bf16), yv[1:2, :].astype(bf16),
         yv[2:3, :].astype(bf16), zero_y], axis=0)
    d = jax.lax.dot_general(
        a_t, b_mat, (((0,), (0,)), ((), ())),
        preferred_element_type=f32)

    s_x = jnp.sum(jnp.maximum(jnp.min(d, axis=1), 0.0))
    tile_miny = jnp.min(d, axis=0, keepdims=True)

    new_miny = jnp.where(i == 0, tile_miny,
                         jnp.minimum(miny_ref[...], tile_miny))
    miny_ref[...] = new_miny

    inc = s_x + jnp.where(i == nb - 1,
                          jnp.sum(jnp.maximum(new_miny, 0.0)), 0.0)
    first = jnp.logical_and(b == 0, i == 0)
    out_ref[0, 0] = jnp.where(first, 0.0, out_ref[0, 0]) + inc


def kernel(x, y):
    B, D, N = x.shape
    M = y.shape[2]

    nb = N // _TN
    out = pl.pallas_call(
        _chamfer_body,
        grid=(B, nb),
        in_specs=[
            pl.BlockSpec((1, D, _TN), lambda b, i: (b, 0, i)),
            pl.BlockSpec((1, D, M), lambda b, i: (b, 0, 0)),
        ],
        out_specs=pl.BlockSpec((1, 1), lambda b, i: (0, 0),
                               memory_space=pltpu.SMEM),
        out_shape=jax.ShapeDtypeStruct((1, 1), jnp.float32),
        scratch_shapes=[pltpu.VMEM((1, M), jnp.float32)],
        compiler_params=pltpu.CompilerParams(
            dimension_semantics=("arbitrary", "arbitrary")),
    )(x, y)

    return out[0, 0] / jnp.float32(B * N)


# speedup vs baseline: 1.3028x; 1.0610x over previous
import jax
import jax.numpy as jnp
from jax.experimental import pallas as pl
from jax.experimental.pallas import tpu as pltpu

_TN = 4096


def _chamfer_body(x_ref, y_ref, out_ref, miny_ref):
    b = pl.program_id(0)
    i = pl.program_id(1)
    nb = pl.num_programs(1)

    xv = x_ref[0]
    yv = y_ref[0]

    f32 = jnp.float32
    bf16 = jnp.bfloat16

    def _rows(v):
        v2 = (v[0:1, :] * v[0:1, :] + v[1:2, :] * v[1:2, :]
              + v[2:3, :] * v[2:3, :])
        v2_hi = v2.astype(bf16)
        v2_lo = (v2 - v2_hi.astype(f32)).astype(bf16)
        ones = jnp.ones_like(v2, dtype=bf16)
        zero = jnp.zeros_like(v2, dtype=bf16)
        return v2_hi, v2_lo, ones, zero

    x2_hi, x2_lo, ones_x, zero_x = _rows(xv)
    y2_hi, y2_lo, ones_y, zero_y = _rows(yv)
    a_t = jnp.concatenate(
        [x2_hi, x2_lo, ones_x, ones_x,
         (-2.0 * xv[0:1, :]).astype(bf16),
         (-2.0 * xv[1:2, :]).astype(bf16),
         (-2.0 * xv[2:3, :]).astype(bf16), zero_x], axis=0)
    b_mat = jnp.concatenate(
        [ones_y, ones_y, y2_hi, y2_lo,
         yv[0:1, :].astype(bf16), yv[1:2, :].astype(bf16),
         yv[2:3, :].astype(bf16), zero_y], axis=0)
    d = jax.lax.dot_general(
        a_t, b_mat, (((0,), (0,)), ((), ())),
        preferred_element_type=f32)

    s_x = jnp.sum(jnp.maximum(jnp.min(d, axis=1), 0.0))
    tile_miny = jnp.min(d, axis=0, keepdims=True)

    new_miny = jnp.where(i == 0, tile_miny,
                         jnp.minimum(miny_ref[...], tile_miny))
    miny_ref[...] = new_miny

    inc = s_x + jnp.where(i == nb - 1,
                          jnp.sum(jnp.maximum(new_miny, 0.0)), 0.0)
    inc = inc * (1.0 / (pl.num_programs(0) * nb * _TN))
    first = jnp.logical_and(b == 0, i == 0)
    out_ref[0, 0] = jnp.where(first, 0.0, out_ref[0, 0]) + inc


def kernel(x, y):
    B, D, N = x.shape
    M = y.shape[2]

    nb = N // _TN
    out = pl.pallas_call(
        _chamfer_body,
        grid=(B, nb),
        in_specs=[
            pl.BlockSpec((1, D, _TN), lambda b, i: (b, 0, i)),
            pl.BlockSpec((1, D, M), lambda b, i: (b, 0, 0)),
        ],
        out_specs=pl.BlockSpec((1, 1), lambda b, i: (0, 0),
                               memory_space=pltpu.SMEM),
        out_shape=jax.ShapeDtypeStruct((1, 1), jnp.float32),
        scratch_shapes=[pltpu.VMEM((1, M), jnp.float32)],
        compiler_params=pltpu.CompilerParams(
            dimension_semantics=("arbitrary", "arbitrary")),
    )(x, y)

    return out[0, 0]
